# baseline (speedup 1.0000x reference)
import functools

import jax
import jax.numpy as jnp
from jax import lax
from jax.experimental import pallas as pl
from jax.experimental.pallas import tpu as pltpu
from jax.experimental.pallas import tpu_sc as plsc

_B, _S, _D = 1, 2048, 768
_DFF = 3072
_E = 8
_DH = 768
_EPS = 1e-5
_TT = 256
_TILE = 128
_CAP = 5120
_NT = _CAP // _TILE
_NW = 32
_TS = _S // _NW
_AS = 2 * _TS

_NEG = -1e30


def _k1_body(x_ref, g_ref, b_ref, w1_ref, b1_ref, w2_ref, b2_ref,
             wr_ref, br_ref, base_ref, h_ref, lg_ref):
    x = x_ref[...]
    mu = jnp.mean(x, axis=-1, keepdims=True)
    var = jnp.mean((x - mu) ** 2, axis=-1, keepdims=True)
    h = (x - mu) / jnp.sqrt(var + _EPS) * g_ref[...] + b_ref[...]
    hb = h.astype(jnp.bfloat16)
    t1 = jnp.dot(hb, w1_ref[...], preferred_element_type=jnp.float32) + b1_ref[...]
    s = jax.nn.gelu(t1)
    sh = jnp.dot(s.astype(jnp.bfloat16), w2_ref[...],
                 preferred_element_type=jnp.float32) + b2_ref[...]
    logits = jnp.dot(hb, wr_ref[...],
                     preferred_element_type=jnp.float32) + br_ref[...]
    base_ref[...] = x + sh
    h_ref[...] = h
    lg_ref[...] = logits


def _k1(xf, ln_g, ln_b, w1b, b_fc1, w2b, b_fc2, wrb, b_router):
    full = lambda shape: pl.BlockSpec(shape, lambda i: (0,) * len(shape))
    row = pl.BlockSpec((_TT, _D), lambda i: (i, 0))
    return pl.pallas_call(
        _k1_body,
        grid=(_S // _TT,),
        in_specs=[
            row, full((_D,)), full((_D,)),
            full((_D, _DFF)), full((_DFF,)),
            full((_DFF, _D)), full((_D,)),
            full((_D, _E)), full((_E,)),
        ],
        out_specs=[row, row, pl.BlockSpec((_TT, _E), lambda i: (i, 0))],
        out_shape=[
            jax.ShapeDtypeStruct((_S, _D), jnp.float32),
            jax.ShapeDtypeStruct((_S, _D), jnp.float32),
            jax.ShapeDtypeStruct((_S, _E), jnp.float32),
        ],
    )(xf, ln_g, ln_b, w1b, b_fc1, w2b, b_fc2, wrb, b_router)


def _lane_extract(v, e):
    lanes = jnp.arange(16, dtype=jnp.int32)
    return jnp.sum(jnp.where(lanes == e, v, 0))


def _k2a_body(lgT_hbm, eid_hbm, rank_hbm, wgt_hbm, hist_hbm,
              lg_v, eid_v, rank_v, wgt_v, stage_v):
    nc = plsc.get_sparse_core_info().num_cores
    wid = lax.axis_index("s") * nc + lax.axis_index("c")
    t0 = wid * _TS

    for e in range(_E):
        pltpu.sync_copy(lgT_hbm.at[e, pl.ds(t0, _TS)], lg_v.at[e])

    for j in range(_TS // 16):
        sl = pl.ds(j * 16, 16)
        le = [lg_v[e, sl] for e in range(_E)]
        m = le[0]
        for e in range(1, _E):
            m = jnp.maximum(m, le[e])
        p = [jnp.exp(v - m) for v in le]
        ssum = p[0]
        for e in range(1, _E):
            ssum = ssum + p[e]
        p = [v / ssum for v in p]

        m1 = p[0]
        i1 = jnp.zeros((16,), jnp.int32)
        for e in range(1, _E):
            upd = p[e] > m1
            m1 = jnp.where(upd, p[e], m1)
            i1 = jnp.where(upd, e, i1)
        m2 = jnp.full((16,), _NEG)
        i2 = jnp.zeros((16,), jnp.int32)
        for e in range(_E):
            pe = jnp.where(i1 == e, _NEG, p[e])
            upd = pe > m2
            m2 = jnp.where(upd, pe, m2)
            i2 = jnp.where(upd, e, i2)
        wsum = m1 + m2
        eid_v[sl] = i1
        eid_v[pl.ds(_TS + j * 16, 16)] = i2
        wgt_v[sl] = m1 / wsum
        wgt_v[pl.ds(_TS + j * 16, 16)] = m2 / wsum

    lanes = jnp.arange(16, dtype=jnp.int32)
    cnt = jnp.zeros((16,), jnp.int32)
    for j in range(_AS // 16):
        sl = pl.ds(j * 16, 16)
        va = eid_v[sl]
        r = jnp.zeros((16,), jnp.int32)
        for e in range(_E):
            msk = va == e
            mi = msk.astype(jnp.int32)
            c = lax.cumsum(mi)
            cnt_e = _lane_extract(cnt, e)
            r = jnp.where(msk, c - 1 + cnt_e, r)
            cnt = cnt + jnp.where(lanes == e, jnp.sum(mi), 0)
        rank_v[sl] = r

    stage_v[...] = cnt
    pltpu.sync_copy(eid_v, eid_hbm.at[wid])
    pltpu.sync_copy(rank_v, rank_hbm.at[wid])
    pltpu.sync_copy(wgt_v, wgt_hbm.at[wid])
    pltpu.sync_copy(stage_v, hist_hbm.at[wid])


def _k2a(logitsT):
    mesh = plsc.VectorSubcoreMesh(core_axis_name="c", subcore_axis_name="s")
    f = pl.kernel(
        _k2a_body,
        mesh=mesh,
        compiler_params=pltpu.CompilerParams(needs_layout_passes=False),
        out_type=[
            jax.ShapeDtypeStruct((_NW, _AS), jnp.int32),
            jax.ShapeDtypeStruct((_NW, _AS), jnp.int32),
            jax.ShapeDtypeStruct((_NW, _AS), jnp.float32),
            jax.ShapeDtypeStruct((_NW, 16), jnp.int32),
        ],
        scratch_types=[
            pltpu.VMEM((_E, _TS), jnp.float32),
            pltpu.VMEM((_AS,), jnp.int32),
            pltpu.VMEM((_AS,), jnp.int32),
            pltpu.VMEM((_AS,), jnp.float32),
            pltpu.VMEM((16,), jnp.int32),
        ],
    )
    return f(logitsT)


def _k2b_body(h_hbm, eid_hbm, rank_hbm, wgt_hbm, hist_hbm,
              gath_hbm, wpad_hbm, pos2_hbm, te_hbm,
              hist_v, eid_v, rank_v, wgt_v, pos_v, rows_v, te_v):
    nc = plsc.get_sparse_core_info().num_cores
    wid = lax.axis_index("s") * nc + lax.axis_index("c")
    t0 = wid * _TS
    lanes = jnp.arange(16, dtype=jnp.int32)

    pltpu.sync_copy(hist_hbm, hist_v)
    pltpu.sync_copy(eid_hbm.at[wid], eid_v)
    pltpu.sync_copy(rank_hbm.at[wid], rank_v)
    pltpu.sync_copy(wgt_hbm.at[wid], wgt_v)

    base = jnp.zeros((16,), jnp.int32)
    tot = jnp.zeros((16,), jnp.int32)
    for w2 in range(_NW):
        hv = hist_v[w2, :]
        tot = tot + hv
        base = base + jnp.where(w2 < wid, hv, 0)
    padded = ((tot + 127) >> 7) << 7
    ends = lax.cumsum(padded)
    starts = ends - padded
    wbase = starts + base

    wb_e = [_lane_extract(wbase, e) for e in range(_E)]

    for j in range(_AS // 16):
        sl = pl.ds(j * 16, 16)
        va = eid_v[sl]
        wb = jnp.zeros((16,), jnp.int32)
        for e in range(_E):
            wb = wb + jnp.where(va == e, wb_e[e], 0)
        pos_v[sl] = wb + rank_v[sl]

    pltpu.sync_copy(pos_v.at[pl.ds(0, _TS)], pos2_hbm.at[0, pl.ds(t0, _TS)])
    pltpu.sync_copy(pos_v.at[pl.ds(_TS, _TS)], pos2_hbm.at[1, pl.ds(t0, _TS)])

    for j in range(_AS // 16):
        sl = pl.ds(j * 16, 16)
        posv = pos_v[sl]
        pltpu.sync_copy(wgt_v.at[sl], wpad_hbm.at[posv])
        tidv = t0 + (j % (_TS // 16)) * 16 + lanes
        pltpu.sync_copy(h_hbm.at[tidv], rows_v)
        pltpu.sync_copy(rows_v, gath_hbm.at[posv])

    @pl.when(wid == 0)
    def _():
        end_e = [_lane_extract(ends, e) for e in range(_E)]
        for v in range(3):
            tstart = (v * 16 + lanes) * _TILE
            acc = jnp.zeros((16,), jnp.int32)
            for e in range(_E):
                acc = acc + (tstart >= end_e[e]).astype(jnp.int32)
            te_v[pl.ds(v * 16, 16)] = jnp.minimum(acc, _E - 1)
        pltpu.sync_copy(te_v, te_hbm)


def _k2b(h, eid, rank, wgt, hist):
    mesh = plsc.VectorSubcoreMesh(core_axis_name="c", subcore_axis_name="s")
    f = pl.kernel(
        _k2b_body,
        mesh=mesh,
        compiler_params=pltpu.CompilerParams(needs_layout_passes=False),
        out_type=[
            jax.ShapeDtypeStruct((_CAP, _D), jnp.float32),
            jax.ShapeDtypeStruct((_CAP,), jnp.float32),
            jax.ShapeDtypeStruct((2, _S), jnp.int32),
            jax.ShapeDtypeStruct((48,), jnp.int32),
        ],
        scratch_types=[
            pltpu.VMEM((_NW, 16), jnp.int32),
            pltpu.VMEM((_AS,), jnp.int32),
            pltpu.VMEM((_AS,), jnp.int32),
            pltpu.VMEM((_AS,), jnp.float32),
            pltpu.VMEM((_AS,), jnp.int32),
            pltpu.VMEM((16, _D), jnp.float32),
            pltpu.VMEM((48,), jnp.int32),
        ],
    )
    return f(h, eid, rank, wgt, hist)


def _k3_body(te_ref, g_ref, we1_ref, be1_ref, we2_ref, be2_ref, wp_ref, o_ref):
    g = g_ref[...].astype(jnp.bfloat16)
    t = jnp.dot(g, we1_ref[0], preferred_element_type=jnp.float32) + be1_ref[0, 0]
    tg = jax.nn.gelu(t)
    o = jnp.dot(tg.astype(jnp.bfloat16), we2_ref[0],
                preferred_element_type=jnp.float32) + be2_ref[0, 0]
    o_ref[...] = o * wp_ref[...][:, None]


def _k3(te_map, gathered, we1b, be1, we2b, be2, wpad):
    grid_spec = pltpu.PrefetchScalarGridSpec(
        num_scalar_prefetch=1,
        grid=(_NT,),
        in_specs=[
            pl.BlockSpec((_TILE, _D), lambda i, te: (i, 0)),
            pl.BlockSpec((1, _D, _DH), lambda i, te: (te[i], 0, 0)),
            pl.BlockSpec((1, 1, _DH), lambda i, te: (te[i], 0, 0)),
            pl.BlockSpec((1, _DH, _D), lambda i, te: (te[i], 0, 0)),
            pl.BlockSpec((1, 1, _D), lambda i, te: (te[i], 0, 0)),
            pl.BlockSpec((_TILE,), lambda i, te: (i,)),
        ],
        out_specs=pl.BlockSpec((_TILE, _D), lambda i, te: (i, 0)),
    )
    return pl.pallas_call(
        _k3_body,
        grid_spec=grid_spec,
        out_shape=jax.ShapeDtypeStruct((_CAP, _D), jnp.float32),
    )(te_map, gathered, we1b, be1, we2b, be2, wpad)


def _k4_body(base_hbm, eo_hbm, pos2_hbm, out_hbm, p0_v, p1_v, acc_v, r0_v, r1_v):
    nc = plsc.get_sparse_core_info().num_cores
    wid = lax.axis_index("s") * nc + lax.axis_index("c")
    t0 = wid * _TS

    pltpu.sync_copy(pos2_hbm.at[0, pl.ds(t0, _TS)], p0_v)
    pltpu.sync_copy(pos2_hbm.at[1, pl.ds(t0, _TS)], p1_v)

    for j in range(_TS // 16):
        sl = pl.ds(j * 16, 16)
        rows = pl.ds(t0 + j * 16, 16)
        pltpu.sync_copy(base_hbm.at[rows, :], acc_v)
        pltpu.sync_copy(eo_hbm.at[p0_v.at[sl]], r0_v)
        pltpu.sync_copy(eo_hbm.at[p1_v.at[sl]], r1_v)

        @pl.loop(0, 16)
        def _(r):
            for c in range(_D // 16):
                cs = pl.ds(c * 16, 16)
                acc_v[r, cs] = acc_v[r, cs] + r0_v[r, cs] + r1_v[r, cs]

        pltpu.sync_copy(acc_v, out_hbm.at[rows, :])


def _k4(base, eo, pos2):
    mesh = plsc.VectorSubcoreMesh(core_axis_name="c", subcore_axis_name="s")
    f = pl.kernel(
        _k4_body,
        mesh=mesh,
        compiler_params=pltpu.CompilerParams(needs_layout_passes=False),
        out_type=jax.ShapeDtypeStruct((_S, _D), jnp.float32),
        scratch_types=[
            pltpu.VMEM((_TS,), jnp.int32),
            pltpu.VMEM((_TS,), jnp.int32),
            pltpu.VMEM((16, _D), jnp.float32),
            pltpu.VMEM((16, _D), jnp.float32),
            pltpu.VMEM((16, _D), jnp.float32),
        ],
    )
    return f(base, eo, pos2)


def kernel(hidden_states, ln_g, ln_b, W_fc1, b_fc1, W_fc2, b_fc2,
           W_router, b_router, We1, be1, We2, be2):
    xf = hidden_states.reshape(_S, _D)
    w1b = W_fc1.astype(jnp.bfloat16)
    w2b = W_fc2.astype(jnp.bfloat16)
    wrb = W_router.astype(jnp.bfloat16)
    we1b = We1.astype(jnp.bfloat16)
    we2b = We2.astype(jnp.bfloat16)

    base, h, logits = _k1(xf, ln_g, ln_b, w1b, b_fc1, w2b, b_fc2,
                          wrb, b_router)
    eid, rank, wgt, hist = _k2a(logits.T)
    gathered, wpad, pos2, te_map = _k2b(h, eid, rank, wgt, hist)
    eo = _k3(te_map, gathered, we1b, be1.reshape(_E, 1, _DH),
             we2b, be2.reshape(_E, 1, _D), wpad)
    out = _k4(base, eo, pos2)
    return out.reshape(_B, _S, _D)


# speedup vs baseline: 1.0095x; 1.0095x over previous
import functools

import jax
import jax.numpy as jnp
from jax import lax
from jax.experimental import pallas as pl
from jax.experimental.pallas import tpu as pltpu
from jax.experimental.pallas import tpu_sc as plsc

_B, _S, _D = 1, 2048, 768
_DFF = 3072
_E = 8
_DH = 768
_EPS = 1e-5
_TT = 256
_TILE = 128
_CAP = 5120
_NT = _CAP // _TILE
_NW = 32
_TS = _S // _NW
_AS = 2 * _TS

_NEG = -1e30


def _k1_body(x_ref, g_ref, b_ref, w1_ref, b1_ref, w2_ref, b2_ref,
             wr_ref, br_ref, base_ref, h_ref, lg_ref):
    x = x_ref[...]
    mu = jnp.mean(x, axis=-1, keepdims=True)
    var = jnp.mean((x - mu) ** 2, axis=-1, keepdims=True)
    h = (x - mu) / jnp.sqrt(var + _EPS) * g_ref[...] + b_ref[...]
    hb = h.astype(jnp.bfloat16)
    t1 = jnp.dot(hb, w1_ref[...], preferred_element_type=jnp.float32) + b1_ref[...]
    s = jax.nn.gelu(t1)
    sh = jnp.dot(s.astype(jnp.bfloat16), w2_ref[...],
                 preferred_element_type=jnp.float32) + b2_ref[...]
    logits = jnp.dot(hb, wr_ref[...],
                     preferred_element_type=jnp.float32) + br_ref[...]
    base_ref[...] = x + sh
    h_ref[...] = h
    lg_ref[...] = logits


def _k1(xf, ln_g, ln_b, w1b, b_fc1, w2b, b_fc2, wrb, b_router):
    full = lambda shape: pl.BlockSpec(shape, lambda i: (0,) * len(shape))
    row = pl.BlockSpec((_TT, _D), lambda i: (i, 0))
    return pl.pallas_call(
        _k1_body,
        grid=(_S // _TT,),
        in_specs=[
            row, full((_D,)), full((_D,)),
            full((_D, _DFF)), full((_DFF,)),
            full((_DFF, _D)), full((_D,)),
            full((_D, _E)), full((_E,)),
        ],
        out_specs=[row, row, pl.BlockSpec((_TT, _E), lambda i: (i, 0))],
        out_shape=[
            jax.ShapeDtypeStruct((_S, _D), jnp.float32),
            jax.ShapeDtypeStruct((_S, _D), jnp.float32),
            jax.ShapeDtypeStruct((_S, _E), jnp.float32),
        ],
    )(xf, ln_g, ln_b, w1b, b_fc1, w2b, b_fc2, wrb, b_router)


def _lane_extract(v, e):
    lanes = jnp.arange(16, dtype=jnp.int32)
    return jnp.sum(jnp.where(lanes == e, v, 0))


def _k2a_body(lgT_hbm, eid_hbm, rank_hbm, wgt_hbm, hist_hbm,
              lg_v, eid_v, rank_v, wgt_v, stage_v):
    nc = plsc.get_sparse_core_info().num_cores
    wid = lax.axis_index("s") * nc + lax.axis_index("c")
    t0 = wid * _TS

    for e in range(_E):
        pltpu.sync_copy(lgT_hbm.at[e, pl.ds(t0, _TS)], lg_v.at[e])

    for j in range(_TS // 16):
        sl = pl.ds(j * 16, 16)
        le = [lg_v[e, sl] for e in range(_E)]
        m = le[0]
        for e in range(1, _E):
            m = jnp.maximum(m, le[e])
        p = [jnp.exp(v - m) for v in le]
        ssum = p[0]
        for e in range(1, _E):
            ssum = ssum + p[e]
        p = [v / ssum for v in p]

        m1 = p[0]
        i1 = jnp.zeros((16,), jnp.int32)
        for e in range(1, _E):
            upd = p[e] > m1
            m1 = jnp.where(upd, p[e], m1)
            i1 = jnp.where(upd, e, i1)
        m2 = jnp.full((16,), _NEG)
        i2 = jnp.zeros((16,), jnp.int32)
        for e in range(_E):
            pe = jnp.where(i1 == e, _NEG, p[e])
            upd = pe > m2
            m2 = jnp.where(upd, pe, m2)
            i2 = jnp.where(upd, e, i2)
        wsum = m1 + m2
        eid_v[sl] = i1
        eid_v[pl.ds(_TS + j * 16, 16)] = i2
        wgt_v[sl] = m1 / wsum
        wgt_v[pl.ds(_TS + j * 16, 16)] = m2 / wsum

    lanes = jnp.arange(16, dtype=jnp.int32)
    cnt = jnp.zeros((16,), jnp.int32)
    for j in range(_AS // 16):
        sl = pl.ds(j * 16, 16)
        va = eid_v[sl]
        r = jnp.zeros((16,), jnp.int32)
        for e in range(_E):
            msk = va == e
            mi = msk.astype(jnp.int32)
            c = lax.cumsum(mi)
            cnt_e = _lane_extract(cnt, e)
            r = jnp.where(msk, c - 1 + cnt_e, r)
            cnt = cnt + jnp.where(lanes == e, jnp.sum(mi), 0)
        rank_v[sl] = r

    stage_v[...] = cnt
    pltpu.sync_copy(eid_v, eid_hbm.at[wid])
    pltpu.sync_copy(rank_v, rank_hbm.at[wid])
    pltpu.sync_copy(wgt_v, wgt_hbm.at[wid])
    pltpu.sync_copy(stage_v, hist_hbm.at[wid])


def _k2a(logitsT):
    mesh = plsc.VectorSubcoreMesh(core_axis_name="c", subcore_axis_name="s")
    f = pl.kernel(
        _k2a_body,
        mesh=mesh,
        compiler_params=pltpu.CompilerParams(needs_layout_passes=False),
        out_type=[
            jax.ShapeDtypeStruct((_NW, _AS), jnp.int32),
            jax.ShapeDtypeStruct((_NW, _AS), jnp.int32),
            jax.ShapeDtypeStruct((_NW, _AS), jnp.float32),
            jax.ShapeDtypeStruct((_NW, 16), jnp.int32),
        ],
        scratch_types=[
            pltpu.VMEM((_E, _TS), jnp.float32),
            pltpu.VMEM((_AS,), jnp.int32),
            pltpu.VMEM((_AS,), jnp.int32),
            pltpu.VMEM((_AS,), jnp.float32),
            pltpu.VMEM((16,), jnp.int32),
        ],
    )
    return f(logitsT)


def _k2b_body(h_hbm, eid_hbm, rank_hbm, wgt_hbm, hist_hbm,
              gath_hbm, wpad_hbm, pos2_hbm, te_hbm,
              hist_v, eid_v, rank_v, wgt_v, pos_v, tid_v, rows_v, te_v):
    nc = plsc.get_sparse_core_info().num_cores
    wid = lax.axis_index("s") * nc + lax.axis_index("c")
    t0 = wid * _TS
    lanes = jnp.arange(16, dtype=jnp.int32)

    pltpu.sync_copy(hist_hbm, hist_v)
    pltpu.sync_copy(eid_hbm.at[wid], eid_v)
    pltpu.sync_copy(rank_hbm.at[wid], rank_v)
    pltpu.sync_copy(wgt_hbm.at[wid], wgt_v)

    base = jnp.zeros((16,), jnp.int32)
    tot = jnp.zeros((16,), jnp.int32)
    for w2 in range(_NW):
        hv = hist_v[w2, :]
        tot = tot + hv
        base = base + jnp.where(w2 < wid, hv, 0)
    padded = ((tot + 127) >> 7) << 7
    ends = lax.cumsum(padded)
    starts = ends - padded
    wbase = starts + base

    wb_e = [_lane_extract(wbase, e) for e in range(_E)]

    for j in range(_AS // 16):
        sl = pl.ds(j * 16, 16)
        va = eid_v[sl]
        wb = jnp.zeros((16,), jnp.int32)
        for e in range(_E):
            wb = wb + jnp.where(va == e, wb_e[e], 0)
        pos_v[sl] = wb + rank_v[sl]

    pltpu.sync_copy(pos_v.at[pl.ds(0, _TS)], pos2_hbm.at[0, pl.ds(t0, _TS)])
    pltpu.sync_copy(pos_v.at[pl.ds(_TS, _TS)], pos2_hbm.at[1, pl.ds(t0, _TS)])

    for j in range(_AS // 16):
        tid_v[pl.ds(j * 16, 16)] = t0 + (j % (_TS // 16)) * 16 + lanes
    pltpu.sync_copy(wgt_v, wpad_hbm.at[pos_v])
    pltpu.sync_copy(h_hbm.at[tid_v], rows_v)
    pltpu.sync_copy(rows_v, gath_hbm.at[pos_v])

    @pl.when(wid == 0)
    def _():
        end_e = [_lane_extract(ends, e) for e in range(_E)]
        for v in range(3):
            tstart = (v * 16 + lanes) * _TILE
            acc = jnp.zeros((16,), jnp.int32)
            for e in range(_E):
                acc = acc + (tstart >= end_e[e]).astype(jnp.int32)
            te_v[pl.ds(v * 16, 16)] = jnp.minimum(acc, _E - 1)
        pltpu.sync_copy(te_v, te_hbm)


def _k2b(h, eid, rank, wgt, hist):
    mesh = plsc.VectorSubcoreMesh(core_axis_name="c", subcore_axis_name="s")
    f = pl.kernel(
        _k2b_body,
        mesh=mesh,
        compiler_params=pltpu.CompilerParams(needs_layout_passes=False),
        out_type=[
            jax.ShapeDtypeStruct((_CAP, _D), jnp.float32),
            jax.ShapeDtypeStruct((_CAP,), jnp.float32),
            jax.ShapeDtypeStruct((2, _S), jnp.int32),
            jax.ShapeDtypeStruct((48,), jnp.int32),
        ],
        scratch_types=[
            pltpu.VMEM((_NW, 16), jnp.int32),
            pltpu.VMEM((_AS,), jnp.int32),
            pltpu.VMEM((_AS,), jnp.int32),
            pltpu.VMEM((_AS,), jnp.float32),
            pltpu.VMEM((_AS,), jnp.int32),
            pltpu.VMEM((_AS,), jnp.int32),
            pltpu.VMEM((_AS, _D), jnp.float32),
            pltpu.VMEM((48,), jnp.int32),
        ],
    )
    return f(h, eid, rank, wgt, hist)


def _k3_body(te_ref, g_ref, we1_ref, be1_ref, we2_ref, be2_ref, wp_ref, o_ref):
    g = g_ref[...].astype(jnp.bfloat16)
    t = jnp.dot(g, we1_ref[0], preferred_element_type=jnp.float32) + be1_ref[0, 0]
    tg = jax.nn.gelu(t)
    o = jnp.dot(tg.astype(jnp.bfloat16), we2_ref[0],
                preferred_element_type=jnp.float32) + be2_ref[0, 0]
    o_ref[...] = o * wp_ref[...][:, None]


def _k3(te_map, gathered, we1b, be1, we2b, be2, wpad):
    grid_spec = pltpu.PrefetchScalarGridSpec(
        num_scalar_prefetch=1,
        grid=(_NT,),
        in_specs=[
            pl.BlockSpec((_TILE, _D), lambda i, te: (i, 0)),
            pl.BlockSpec((1, _D, _DH), lambda i, te: (te[i], 0, 0)),
            pl.BlockSpec((1, 1, _DH), lambda i, te: (te[i], 0, 0)),
            pl.BlockSpec((1, _DH, _D), lambda i, te: (te[i], 0, 0)),
            pl.BlockSpec((1, 1, _D), lambda i, te: (te[i], 0, 0)),
            pl.BlockSpec((_TILE,), lambda i, te: (i,)),
        ],
        out_specs=pl.BlockSpec((_TILE, _D), lambda i, te: (i, 0)),
    )
    return pl.pallas_call(
        _k3_body,
        grid_spec=grid_spec,
        out_shape=jax.ShapeDtypeStruct((_CAP, _D), jnp.float32),
    )(te_map, gathered, we1b, be1, we2b, be2, wpad)


_HT = _TS // 2


def _k4_body(base_hbm, eo_hbm, pos2_hbm, out_hbm, p0_v, p1_v,
             acc_v, r0_v, r1_v, sema, semb, semo):
    nc = plsc.get_sparse_core_info().num_cores
    wid = lax.axis_index("s") * nc + lax.axis_index("c")
    t0 = wid * _TS

    pltpu.sync_copy(pos2_hbm.at[0, pl.ds(t0, _TS)], p0_v)
    pltpu.sync_copy(pos2_hbm.at[1, pl.ds(t0, _TS)], p1_v)

    prev = None
    for half in range(2):
        sl = pl.ds(half * _HT, _HT)
        rows = pl.ds(t0 + half * _HT, _HT)
        c0 = pltpu.async_copy(eo_hbm.at[p0_v.at[sl]], r0_v, semb)
        c1 = pltpu.async_copy(eo_hbm.at[p1_v.at[sl]], r1_v, semb)
        if prev is not None:
            prev.wait()
        ca = pltpu.async_copy(base_hbm.at[rows, :], acc_v, sema)
        ca.wait()
        c0.wait()
        c1.wait()

        @pl.loop(0, _HT)
        def _(r):
            for c in range(_D // 16):
                cs = pl.ds(c * 16, 16)
                acc_v[r, cs] = acc_v[r, cs] + r0_v[r, cs] + r1_v[r, cs]

        prev = pltpu.async_copy(acc_v, out_hbm.at[rows, :], semo)
    prev.wait()


def _k4(base, eo, pos2):
    mesh = plsc.VectorSubcoreMesh(core_axis_name="c", subcore_axis_name="s")
    f = pl.kernel(
        _k4_body,
        mesh=mesh,
        compiler_params=pltpu.CompilerParams(needs_layout_passes=False),
        out_type=jax.ShapeDtypeStruct((_S, _D), jnp.float32),
        scratch_types=[
            pltpu.VMEM((_TS,), jnp.int32),
            pltpu.VMEM((_TS,), jnp.int32),
            pltpu.VMEM((_HT, _D), jnp.float32),
            pltpu.VMEM((_HT, _D), jnp.float32),
            pltpu.VMEM((_HT, _D), jnp.float32),
            pltpu.SemaphoreType.DMA,
            pltpu.SemaphoreType.DMA,
            pltpu.SemaphoreType.DMA,
        ],
    )
    return f(base, eo, pos2)


def kernel(hidden_states, ln_g, ln_b, W_fc1, b_fc1, W_fc2, b_fc2,
           W_router, b_router, We1, be1, We2, be2):
    xf = hidden_states.reshape(_S, _D)
    w1b = W_fc1.astype(jnp.bfloat16)
    w2b = W_fc2.astype(jnp.bfloat16)
    wrb = W_router.astype(jnp.bfloat16)
    we1b = We1.astype(jnp.bfloat16)
    we2b = We2.astype(jnp.bfloat16)

    base, h, logits = _k1(xf, ln_g, ln_b, w1b, b_fc1, w2b, b_fc2,
                          wrb, b_router)
    eid, rank, wgt, hist = _k2a(logits.T)
    gathered, wpad, pos2, te_map = _k2b(h, eid, rank, wgt, hist)
    eo = _k3(te_map, gathered, we1b, be1.reshape(_E, 1, _DH),
             we2b, be2.reshape(_E, 1, _D), wpad)
    out = _k4(base, eo, pos2)
    return out.reshape(_B, _S, _D)


# speedup vs baseline: 1.1950x; 1.1838x over previous
import functools

import jax
import jax.numpy as jnp
from jax import lax
from jax.experimental import pallas as pl
from jax.experimental.pallas import tpu as pltpu
from jax.experimental.pallas import tpu_sc as plsc

_B, _S, _D = 1, 2048, 768
_DFF = 3072
_E = 8
_DH = 768
_EPS = 1e-5
_TT = 256
_TILE = 128
_CAP = 5120
_NT = _CAP // _TILE
_NW = 32
_TS = _S // _NW
_AS = 2 * _TS

_NEG = -1e30


def _k1_body(x_ref, g_ref, b_ref, w1_ref, b1_ref, w2_ref, b2_ref,
             wr_ref, br_ref, base_ref, h_ref, lg_ref):
    x = x_ref[...]
    mu = jnp.mean(x, axis=-1, keepdims=True)
    var = jnp.mean((x - mu) ** 2, axis=-1, keepdims=True)
    h = (x - mu) / jnp.sqrt(var + _EPS) * g_ref[...] + b_ref[...]
    hb = h.astype(jnp.bfloat16)
    t1 = jnp.dot(hb, w1_ref[...], preferred_element_type=jnp.float32) + b1_ref[...]
    s = jax.nn.gelu(t1)
    sh = jnp.dot(s.astype(jnp.bfloat16), w2_ref[...],
                 preferred_element_type=jnp.float32) + b2_ref[...]
    logits = jnp.dot(hb, wr_ref[...],
                     preferred_element_type=jnp.float32) + br_ref[...]
    base_ref[...] = x + sh
    h_ref[...] = h
    lg_ref[...] = logits


def _k1(xf, ln_g, ln_b, w1b, b_fc1, w2b, b_fc2, wrb, b_router):
    full = lambda shape: pl.BlockSpec(shape, lambda i: (0,) * len(shape))
    row = pl.BlockSpec((_TT, _D), lambda i: (i, 0))
    return pl.pallas_call(
        _k1_body,
        grid=(_S // _TT,),
        in_specs=[
            row, full((_D,)), full((_D,)),
            full((_D, _DFF)), full((_DFF,)),
            full((_DFF, _D)), full((_D,)),
            full((_D, _E)), full((_E,)),
        ],
        out_specs=[row, row, pl.BlockSpec((_TT, _E), lambda i: (i, 0))],
        out_shape=[
            jax.ShapeDtypeStruct((_S, _D), jnp.float32),
            jax.ShapeDtypeStruct((_S, _D), jnp.float32),
            jax.ShapeDtypeStruct((_S, _E), jnp.float32),
        ],
    )(xf, ln_g, ln_b, w1b, b_fc1, w2b, b_fc2, wrb, b_router)


def _lane_extract(v, e):
    lanes = jnp.arange(16, dtype=jnp.int32)
    return jnp.sum(jnp.where(lanes == e, v, 0))


def _k2a_body(lgT_hbm, eid_hbm, rank_hbm, wgt_hbm, hist_hbm,
              lg_v, eid_v, rank_v, wgt_v, stage_v):
    nc = plsc.get_sparse_core_info().num_cores
    wid = lax.axis_index("s") * nc + lax.axis_index("c")
    t0 = wid * _TS

    for e in range(_E):
        pltpu.sync_copy(lgT_hbm.at[e, pl.ds(t0, _TS)], lg_v.at[e])

    for j in range(_TS // 16):
        sl = pl.ds(j * 16, 16)
        le = [lg_v[e, sl] for e in range(_E)]
        m = le[0]
        for e in range(1, _E):
            m = jnp.maximum(m, le[e])
        p = [jnp.exp(v - m) for v in le]
        ssum = p[0]
        for e in range(1, _E):
            ssum = ssum + p[e]
        p = [v / ssum for v in p]

        m1 = p[0]
        i1 = jnp.zeros((16,), jnp.int32)
        for e in range(1, _E):
            upd = p[e] > m1
            m1 = jnp.where(upd, p[e], m1)
            i1 = jnp.where(upd, e, i1)
        m2 = jnp.full((16,), _NEG)
        i2 = jnp.zeros((16,), jnp.int32)
        for e in range(_E):
            pe = jnp.where(i1 == e, _NEG, p[e])
            upd = pe > m2
            m2 = jnp.where(upd, pe, m2)
            i2 = jnp.where(upd, e, i2)
        wsum = m1 + m2
        eid_v[sl] = i1
        eid_v[pl.ds(_TS + j * 16, 16)] = i2
        wgt_v[sl] = m1 / wsum
        wgt_v[pl.ds(_TS + j * 16, 16)] = m2 / wsum

    lanes = jnp.arange(16, dtype=jnp.int32)
    cnt = jnp.zeros((16,), jnp.int32)
    for j in range(_AS // 16):
        sl = pl.ds(j * 16, 16)
        va = eid_v[sl]
        r = jnp.zeros((16,), jnp.int32)
        for e in range(_E):
            msk = va == e
            mi = msk.astype(jnp.int32)
            c = lax.cumsum(mi)
            cnt_e = _lane_extract(cnt, e)
            r = jnp.where(msk, c - 1 + cnt_e, r)
            cnt = cnt + jnp.where(lanes == e, jnp.sum(mi), 0)
        rank_v[sl] = r

    stage_v[...] = cnt
    pltpu.sync_copy(eid_v, eid_hbm.at[wid])
    pltpu.sync_copy(rank_v, rank_hbm.at[wid])
    pltpu.sync_copy(wgt_v, wgt_hbm.at[wid])
    pltpu.sync_copy(stage_v, hist_hbm.at[wid])


def _k2a(logitsT):
    mesh = plsc.VectorSubcoreMesh(core_axis_name="c", subcore_axis_name="s")
    f = pl.kernel(
        _k2a_body,
        mesh=mesh,
        compiler_params=pltpu.CompilerParams(needs_layout_passes=False),
        out_type=[
            jax.ShapeDtypeStruct((_NW, _AS), jnp.int32),
            jax.ShapeDtypeStruct((_NW, _AS), jnp.int32),
            jax.ShapeDtypeStruct((_NW, _AS), jnp.float32),
            jax.ShapeDtypeStruct((_NW, 16), jnp.int32),
        ],
        scratch_types=[
            pltpu.VMEM((_E, _TS), jnp.float32),
            pltpu.VMEM((_AS,), jnp.int32),
            pltpu.VMEM((_AS,), jnp.int32),
            pltpu.VMEM((_AS,), jnp.float32),
            pltpu.VMEM((16,), jnp.int32),
        ],
    )
    return f(logitsT)


def _k2b_body(h_hbm, eid_hbm, rank_hbm, hist_hbm,
              gath_hbm, pos2_hbm, te_hbm,
              hist_v, eid_v, rank_v, pos_v, tid_v, rows_v, te_v, sem_in):
    nc = plsc.get_sparse_core_info().num_cores
    wid = lax.axis_index("s") * nc + lax.axis_index("c")
    t0 = wid * _TS
    lanes = jnp.arange(16, dtype=jnp.int32)

    ch = pltpu.async_copy(hist_hbm, hist_v, sem_in)
    ce = pltpu.async_copy(eid_hbm.at[wid], eid_v, sem_in)
    cr = pltpu.async_copy(rank_hbm.at[wid], rank_v, sem_in)
    ch.wait()
    ce.wait()
    cr.wait()

    base = jnp.zeros((16,), jnp.int32)
    tot = jnp.zeros((16,), jnp.int32)
    for w2 in range(_NW):
        hv = hist_v[w2, :]
        tot = tot + hv
        base = base + jnp.where(w2 < wid, hv, 0)
    padded = ((tot + 127) >> 7) << 7
    ends = lax.cumsum(padded)
    starts = ends - padded
    wbase = starts + base

    wb_e = [_lane_extract(wbase, e) for e in range(_E)]

    for j in range(_AS // 16):
        sl = pl.ds(j * 16, 16)
        va = eid_v[sl]
        wb = jnp.zeros((16,), jnp.int32)
        for e in range(_E):
            wb = wb + jnp.where(va == e, wb_e[e], 0)
        pos_v[sl] = wb + rank_v[sl]

    pltpu.sync_copy(pos_v.at[pl.ds(0, _TS)], pos2_hbm.at[0, pl.ds(t0, _TS)])
    pltpu.sync_copy(pos_v.at[pl.ds(_TS, _TS)], pos2_hbm.at[1, pl.ds(t0, _TS)])

    for j in range(_AS // 16):
        tid_v[pl.ds(j * 16, 16)] = t0 + (j % (_TS // 16)) * 16 + lanes
    pltpu.sync_copy(h_hbm.at[tid_v], rows_v)
    pltpu.sync_copy(rows_v, gath_hbm.at[pos_v])

    @pl.when(wid == 0)
    def _():
        end_e = [_lane_extract(ends, e) for e in range(_E)]
        for v in range(3):
            tstart = (v * 16 + lanes) * _TILE
            acc = jnp.zeros((16,), jnp.int32)
            for e in range(_E):
                acc = acc + (tstart >= end_e[e]).astype(jnp.int32)
            te_v[pl.ds(v * 16, 16)] = jnp.minimum(acc, _E - 1)
        pltpu.sync_copy(te_v, te_hbm)


def _k2b(h, eid, rank, hist):
    mesh = plsc.VectorSubcoreMesh(core_axis_name="c", subcore_axis_name="s")
    f = pl.kernel(
        _k2b_body,
        mesh=mesh,
        compiler_params=pltpu.CompilerParams(needs_layout_passes=False),
        out_type=[
            jax.ShapeDtypeStruct((_CAP, _D), jnp.float32),
            jax.ShapeDtypeStruct((2, _S), jnp.int32),
            jax.ShapeDtypeStruct((48,), jnp.int32),
        ],
        scratch_types=[
            pltpu.VMEM((_NW, 16), jnp.int32),
            pltpu.VMEM((_AS,), jnp.int32),
            pltpu.VMEM((_AS,), jnp.int32),
            pltpu.VMEM((_AS,), jnp.int32),
            pltpu.VMEM((_AS,), jnp.int32),
            pltpu.VMEM((_AS, _D), jnp.float32),
            pltpu.VMEM((48,), jnp.int32),
            pltpu.SemaphoreType.DMA,
        ],
    )
    return f(h, eid, rank, hist)


def _k3_body(te_ref, g_ref, we1_ref, be1_ref, we2_ref, be2_ref, o_ref):
    g = g_ref[...].astype(jnp.bfloat16)
    t = jnp.dot(g, we1_ref[0], preferred_element_type=jnp.float32) + be1_ref[0, 0]
    tg = jax.nn.gelu(t)
    o_ref[...] = jnp.dot(tg.astype(jnp.bfloat16), we2_ref[0],
                         preferred_element_type=jnp.float32) + be2_ref[0, 0]


def _k3(te_map, gathered, we1b, be1, we2b, be2):
    grid_spec = pltpu.PrefetchScalarGridSpec(
        num_scalar_prefetch=1,
        grid=(_NT,),
        in_specs=[
            pl.BlockSpec((_TILE, _D), lambda i, te: (i, 0)),
            pl.BlockSpec((1, _D, _DH), lambda i, te: (te[i], 0, 0)),
            pl.BlockSpec((1, 1, _DH), lambda i, te: (te[i], 0, 0)),
            pl.BlockSpec((1, _DH, _D), lambda i, te: (te[i], 0, 0)),
            pl.BlockSpec((1, 1, _D), lambda i, te: (te[i], 0, 0)),
        ],
        out_specs=pl.BlockSpec((_TILE, _D), lambda i, te: (i, 0)),
    )
    return pl.pallas_call(
        _k3_body,
        grid_spec=grid_spec,
        out_shape=jax.ShapeDtypeStruct((_CAP, _D), jnp.float32),
    )(te_map, gathered, we1b, be1, we2b, be2)


_HT = _TS // 2


def _k4_body(base_hbm, eo_hbm, pos2_hbm, wgt_hbm, out_hbm, p0_v, p1_v, wv_v,
             acc_v, r0_v, r1_v, sema, semb, semo):
    nc = plsc.get_sparse_core_info().num_cores
    wid = lax.axis_index("s") * nc + lax.axis_index("c")
    t0 = wid * _TS
    lanes = jnp.arange(16, dtype=jnp.int32)

    cw = pltpu.async_copy(wgt_hbm.at[wid], wv_v, sema)
    pltpu.sync_copy(pos2_hbm.at[0, pl.ds(t0, _TS)], p0_v)
    pltpu.sync_copy(pos2_hbm.at[1, pl.ds(t0, _TS)], p1_v)
    cw.wait()

    prev = None
    for half in range(2):
        sl = pl.ds(half * _HT, _HT)
        rows = pl.ds(t0 + half * _HT, _HT)
        c0 = pltpu.async_copy(eo_hbm.at[p0_v.at[sl]], r0_v, semb)
        c1 = pltpu.async_copy(eo_hbm.at[p1_v.at[sl]], r1_v, semb)
        if prev is not None:
            prev.wait()
        ca = pltpu.async_copy(base_hbm.at[rows, :], acc_v, sema)
        ca.wait()
        c0.wait()
        c1.wait()

        @pl.loop(0, _HT)
        def _(r):
            tl = half * _HT + r
            vbase = (tl // 16) * 16
            lane = tl - vbase
            vec0 = wv_v[pl.ds(vbase, 16)]
            vec1 = wv_v[pl.ds(_TS + vbase, 16)]
            zero = jnp.zeros((16,), jnp.float32)
            w0 = zero + jnp.sum(jnp.where(lanes == lane, vec0, 0.0))
            w1 = zero + jnp.sum(jnp.where(lanes == lane, vec1, 0.0))
            for c in range(_D // 16):
                cs = pl.ds(c * 16, 16)
                acc_v[r, cs] = (acc_v[r, cs] + w0 * r0_v[r, cs]
                                + w1 * r1_v[r, cs])

        prev = pltpu.async_copy(acc_v, out_hbm.at[rows, :], semo)
    prev.wait()


def _k4(base, eo, pos2, wgt):
    mesh = plsc.VectorSubcoreMesh(core_axis_name="c", subcore_axis_name="s")
    f = pl.kernel(
        _k4_body,
        mesh=mesh,
        compiler_params=pltpu.CompilerParams(needs_layout_passes=False),
        out_type=jax.ShapeDtypeStruct((_S, _D), jnp.float32),
        scratch_types=[
            pltpu.VMEM((_TS,), jnp.int32),
            pltpu.VMEM((_TS,), jnp.int32),
            pltpu.VMEM((_AS,), jnp.float32),
            pltpu.VMEM((_HT, _D), jnp.float32),
            pltpu.VMEM((_HT, _D), jnp.float32),
            pltpu.VMEM((_HT, _D), jnp.float32),
            pltpu.SemaphoreType.DMA,
            pltpu.SemaphoreType.DMA,
            pltpu.SemaphoreType.DMA,
        ],
    )
    return f(base, eo, pos2, wgt)


def kernel(hidden_states, ln_g, ln_b, W_fc1, b_fc1, W_fc2, b_fc2,
           W_router, b_router, We1, be1, We2, be2):
    xf = hidden_states.reshape(_S, _D)
    w1b = W_fc1.astype(jnp.bfloat16)
    w2b = W_fc2.astype(jnp.bfloat16)
    wrb = W_router.astype(jnp.bfloat16)
    we1b = We1.astype(jnp.bfloat16)
    we2b = We2.astype(jnp.bfloat16)

    base, h, logits = _k1(xf, ln_g, ln_b, w1b, b_fc1, w2b, b_fc2,
                          wrb, b_router)
    eid, rank, wgt, hist = _k2a(logits.T)
    gathered, pos2, te_map = _k2b(h, eid, rank, hist)
    eo = _k3(te_map, gathered, we1b, be1.reshape(_E, 1, _DH),
             we2b, be2.reshape(_E, 1, _D))
    out = _k4(base, eo, pos2, wgt)
    return out.reshape(_B, _S, _D)
